# baseline (speedup 1.0000x reference)
import functools

import jax
import jax.numpy as jnp
from jax.experimental import pallas as pl
from jax.experimental.pallas import tpu as pltpu

NEG_INF = float("-inf")


def _gelu(z):
    return z * 0.5 * (1.0 + jax.lax.erf(z * 0.7071067811865476))


def _main_body(eid_ref, pperm_ref, h_ref,
               wc1_ref, bc1_ref, wc2_ref, bc2_ref,
               wf1_ref, bf1_ref, wf2_ref, bf2_ref,
               sc_out, sf_out,
               acc_c, acc_f,
               *, bt, n_g, n_h, chunk):
    g = pl.program_id(0)
    hs = pl.program_id(1)
    hh = h_ref[...].astype(jnp.bfloat16)

    zc = jax.lax.dot_general(
        hh, wc1_ref[...], (((1,), (1,)), ((), ())),
        preferred_element_type=jnp.float32)
    zc = _gelu(zc + bc1_ref[...]).astype(jnp.bfloat16)
    pc = jax.lax.dot_general(
        zc, wc2_ref[...], (((1,), (1,)), ((), ())),
        preferred_element_type=jnp.float32)

    zf = jax.lax.dot_general(
        hh, wf1_ref[0], (((1,), (1,)), ((), ())),
        preferred_element_type=jnp.float32)
    zf = _gelu(zf + bf1_ref[0]).astype(jnp.bfloat16)
    pf = jax.lax.dot_general(
        zf, wf2_ref[0], (((1,), (1,)), ((), ())),
        preferred_element_type=jnp.float32)

    @pl.when(hs == 0)
    def _():
        acc_c[...] = pc
        acc_f[...] = pf

    @pl.when(hs != 0)
    def _():
        acc_c[...] += pc
        acc_f[...] += pf

    @pl.when(hs == n_h - 1)
    def _():
        sc_out[...] = (acc_c[...] + bc2_ref[...]).astype(jnp.bfloat16)
        sf_out[...] = (acc_f[...] + bf2_ref[0]).astype(jnp.bfloat16)


def _epi_body(invpos_ref, labels_ref, sc_ref, sf_ref,
              coarse_out, fine_out, flat_out, *, gbt, nf0, nf1):
    ip = invpos_ref[...]
    iota = jax.lax.broadcasted_iota(jnp.int32, (ip.shape[0], gbt), 1)
    pmat = (ip == iota).astype(jnp.bfloat16)
    coarse_out[...] = jax.lax.dot_general(
        pmat, sc_ref[...], (((1,), (0,)), ((), ())),
        preferred_element_type=jnp.float32)
    logits = jax.lax.dot_general(
        pmat, sf_ref[...], (((1,), (0,)), ((), ())),
        preferred_element_type=jnp.float32)
    mask = labels_ref[...] == 0
    neg = jnp.float32(NEG_INF)
    col = jax.lax.broadcasted_iota(jnp.int32, logits.shape, 1)
    fine_out[...] = jnp.where(mask & (col >= nf0), neg, logits)
    flat_out[...] = jnp.concatenate(
        [jnp.where(mask, logits[:, :nf0], neg),
         jnp.where(mask, neg, logits)], axis=1)


def kernel(h, coarse_labels, Wc1, bc1, Wc2, bc2,
           Wf0_1, bf0_1, Wf0_2, bf0_2, Wf1_1, bf1_1, Wf1_2, bf1_2):
    B, IN = h.shape
    H = Wc1.shape[0]
    NC = Wc2.shape[0]
    NF0 = Wf0_2.shape[0]
    NF1 = Wf1_2.shape[0]
    bt = min(512, B)
    hc = min(512, H)
    n_h = H // hc
    n_g = B // bt + 1
    gbt = n_g * bt
    chunk = bt // n_h

    labels = coarse_labels.astype(jnp.int32)
    m0 = labels == 0
    c0 = jnp.cumsum(m0.astype(jnp.int32))
    count0 = c0[-1]
    c0ex = c0 - m0.astype(jnp.int32)
    idx = jnp.arange(B, dtype=jnp.int32)
    t1 = (count0 + bt - 1) // bt
    pos = jnp.where(m0, c0ex, t1 * bt + idx - c0ex)
    pperm = jnp.zeros((gbt,), jnp.int32).at[pos].set(idx)
    eid = (jnp.arange(n_g, dtype=jnp.int32) >= t1).astype(jnp.int32)
    invpos = pos.reshape(B, 1)
    labels2 = labels.reshape(B, 1)

    bf = jnp.bfloat16
    wc1b = Wc1.astype(bf)
    bc1r = bc1.reshape(1, H)
    wc2b = Wc2.astype(bf)
    bc2r = bc2.reshape(1, NC)
    wf1_st = jnp.stack([Wf0_1, Wf1_1]).astype(bf)
    bf1_st = jnp.stack([bf0_1, bf1_1]).reshape(2, 1, H)
    wf2_st = jnp.stack([
        jnp.concatenate([Wf0_2, jnp.zeros((NF1 - NF0, H), Wf0_2.dtype)], 0),
        Wf1_2]).astype(bf)
    bf2_st = jnp.stack([
        jnp.concatenate([bf0_2, jnp.zeros((NF1 - NF0,), bf0_2.dtype)]),
        bf1_2]).reshape(2, 1, NF1)

    grid_spec = pltpu.PrefetchScalarGridSpec(
        num_scalar_prefetch=2,
        grid=(n_g, n_h),
        in_specs=[
            pl.BlockSpec((bt, IN), lambda g, hs, e, p: (g, 0)),
            pl.BlockSpec((hc, IN), lambda g, hs, e, p: (hs, 0)),
            pl.BlockSpec((1, hc), lambda g, hs, e, p: (0, hs)),
            pl.BlockSpec((NC, hc), lambda g, hs, e, p: (0, hs)),
            pl.BlockSpec((1, NC), lambda g, hs, e, p: (0, 0)),
            pl.BlockSpec((1, hc, IN), lambda g, hs, e, p: (e[g], hs, 0)),
            pl.BlockSpec((1, 1, hc), lambda g, hs, e, p: (e[g], 0, hs)),
            pl.BlockSpec((1, NF1, hc), lambda g, hs, e, p: (e[g], 0, hs)),
            pl.BlockSpec((1, 1, NF1), lambda g, hs, e, p: (e[g], 0, 0)),
        ],
        out_specs=[
            pl.BlockSpec((bt, NC), lambda g, hs, e, p: (g, 0)),
            pl.BlockSpec((bt, NF1), lambda g, hs, e, p: (g, 0)),
        ],
        scratch_shapes=[
            pltpu.VMEM((bt, NC), jnp.float32),
            pltpu.VMEM((bt, NF1), jnp.float32),
        ],
    )
    sc, sf = pl.pallas_call(
        functools.partial(_main_body, bt=bt, n_g=n_g, n_h=n_h, chunk=chunk),
        grid_spec=grid_spec,
        out_shape=[
            jax.ShapeDtypeStruct((gbt, NC), bf),
            jax.ShapeDtypeStruct((gbt, NF1), bf),
        ],
    )(eid, pperm, jnp.take(h, pperm, axis=0),
      wc1b, bc1r, wc2b, bc2r, wf1_st, bf1_st, wf2_st, bf2_st)

    bt2 = min(512, B)
    coarse, fine, flat = pl.pallas_call(
        functools.partial(_epi_body, gbt=gbt, nf0=NF0, nf1=NF1),
        grid=(B // bt2,),
        in_specs=[
            pl.BlockSpec((bt2, 1), lambda b: (b, 0)),
            pl.BlockSpec((bt2, 1), lambda b: (b, 0)),
            pl.BlockSpec((gbt, NC), lambda b: (0, 0)),
            pl.BlockSpec((gbt, NF1), lambda b: (0, 0)),
        ],
        out_specs=[
            pl.BlockSpec((bt2, NC), lambda b: (b, 0)),
            pl.BlockSpec((bt2, NF1), lambda b: (b, 0)),
            pl.BlockSpec((bt2, NF0 + NF1), lambda b: (b, 0)),
        ],
        out_shape=[
            jax.ShapeDtypeStruct((B, NC), jnp.float32),
            jax.ShapeDtypeStruct((B, NF1), jnp.float32),
            jax.ShapeDtypeStruct((B, NF0 + NF1), jnp.float32),
        ],
    )(invpos, labels2, sc, sf)
    return (coarse, fine, flat)


# speedup vs baseline: 1.3187x; 1.3187x over previous
import functools

import jax
import jax.numpy as jnp
from jax.experimental import pallas as pl
from jax.experimental.pallas import tpu as pltpu

NEG_INF = float("-inf")


def _fused_body(labels_ref, h_ref, w1_ref, b1_ref, w2_ref, b2_ref,
                coarse_out, fine_out, flat_out, acc,
                *, n_h, nf0, nf1):
    hstep = pl.program_id(1)
    hh = h_ref[...].astype(jnp.bfloat16)

    z = jax.lax.dot_general(
        hh, w1_ref[0], (((1,), (1,)), ((), ())),
        preferred_element_type=jnp.float32)
    z = z + b1_ref[0]
    z = (z * 0.5 * (1.0 + jax.lax.erf(z * 0.7071067811865476))
         ).astype(jnp.bfloat16)
    part = jax.lax.dot_general(
        z, w2_ref[0], (((1,), (0,)), ((), ())),
        preferred_element_type=jnp.float32)

    @pl.when(hstep == 0)
    def _():
        acc[...] = part

    @pl.when(hstep != 0)
    def _():
        acc[...] += part

    @pl.when(hstep == n_h - 1)
    def _():
        res = acc[...] + b2_ref[...]
        nc = coarse_out.shape[1]
        coarse_out[...] = res[:, :nc]
        l0 = res[:, nf1 - nf0:nf1]
        l1 = res[:, nf1:nf1 + nf1]
        mask = labels_ref[...] == 0
        neg = jnp.float32(NEG_INF)
        pad0 = jnp.concatenate(
            [l0, jnp.full((l0.shape[0], nf1 - nf0), neg, jnp.float32)], axis=1)
        fine_out[...] = jnp.where(mask, pad0, l1)
        flat_out[...] = jnp.concatenate(
            [jnp.where(mask, l0, neg), jnp.where(mask, neg, l1)], axis=1)


def kernel(h, coarse_labels, Wc1, bc1, Wc2, bc2,
           Wf0_1, bf0_1, Wf0_2, bf0_2, Wf1_1, bf1_1, Wf1_2, bf1_2):
    B, IN = h.shape
    H = Wc1.shape[0]
    NC = Wc2.shape[0]
    NF0 = Wf0_2.shape[0]
    NF1 = Wf1_2.shape[0]
    bt = min(512, B)
    hc = min(512, H)
    n_b = B // bt
    n_h = H // hc
    npack = NF1 + NF1

    bf = jnp.bfloat16
    w1_all = jnp.concatenate([
        Wc1.reshape(n_h, hc, IN),
        Wf0_1.reshape(n_h, hc, IN),
        Wf1_1.reshape(n_h, hc, IN)], axis=1).astype(bf)
    b1_all = jnp.concatenate([
        bc1.reshape(n_h, 1, hc),
        bf0_1.reshape(n_h, 1, hc),
        bf1_1.reshape(n_h, 1, hc)], axis=1).reshape(n_h, 1, 3 * hc)
    zpad = functools.partial(jnp.zeros, dtype=jnp.float32)
    w2_all = jnp.concatenate([
        jnp.concatenate([Wc2.T.reshape(n_h, hc, NC),
                         zpad((n_h, hc, NF1 - NF0 - NC)),
                         zpad((n_h, hc, NF0 + NF1))], axis=2),
        jnp.concatenate([zpad((n_h, hc, NF1 - NF0)),
                         Wf0_2.T.reshape(n_h, hc, NF0),
                         zpad((n_h, hc, NF1))], axis=2),
        jnp.concatenate([zpad((n_h, hc, NF1)),
                         Wf1_2.T.reshape(n_h, hc, NF1)], axis=2),
    ], axis=1).astype(bf)
    b2_all = jnp.concatenate([
        bc2, jnp.zeros((NF1 - NF0 - NC,), jnp.float32), bf0_2, bf1_2,
    ]).reshape(1, npack)
    labels2 = coarse_labels.reshape(B, 1)

    in_specs = [
        pl.BlockSpec((bt, 1), lambda b, hs: (b, 0)),
        pl.BlockSpec((bt, IN), lambda b, hs: (b, 0)),
        pl.BlockSpec((1, 3 * hc, IN), lambda b, hs: (hs, 0, 0)),
        pl.BlockSpec((1, 1, 3 * hc), lambda b, hs: (hs, 0, 0)),
        pl.BlockSpec((1, 3 * hc, npack), lambda b, hs: (hs, 0, 0)),
        pl.BlockSpec((1, npack), lambda b, hs: (0, 0)),
    ]

    def out_spec(n):
        return pl.BlockSpec((bt, n), lambda b, hs: (b, 0))

    coarse, fine, flat = pl.pallas_call(
        functools.partial(_fused_body, n_h=n_h, nf0=NF0, nf1=NF1),
        grid=(n_b, n_h),
        in_specs=in_specs,
        out_specs=[out_spec(NC), out_spec(NF1), out_spec(NF0 + NF1)],
        out_shape=[
            jax.ShapeDtypeStruct((B, NC), jnp.float32),
            jax.ShapeDtypeStruct((B, NF1), jnp.float32),
            jax.ShapeDtypeStruct((B, NF0 + NF1), jnp.float32),
        ],
        scratch_shapes=[
            pltpu.VMEM((bt, npack), jnp.float32),
        ],
    )(labels2, h, w1_all, b1_all, w2_all, b2_all)
    return (coarse, fine, flat)
